# baseline (speedup 1.0000x reference)
import functools

import jax
import jax.numpy as jnp
from jax import lax
from jax.experimental import pallas as pl
from jax.experimental.pallas import tpu as pltpu
from jax.experimental.pallas import tpu_sc as plsc

_N = 10000
_E = 320000
_EL = 320000

_NC = 2
_NS = 16
_NW = _NC * _NS

_SUB = 80
_EPW = _E // _NW
_NSUB = _EPW // _SUB
_WB = 624
_WBL = _N - 15 * _WB
_NB = 4
_NBI = 25
_NBLK = _NSUB // _NBI


def _mesh():
    return plsc.VectorSubcoreMesh(
        core_axis_name="c", subcore_axis_name="s",
        num_cores=_NC, num_subcores=_NS)


def _segsum(tab, src3, dst3, zeros):
    D = tab.shape[1]

    def body(tab_ref, src_ref, dst_ref, zeros_ref, out_ref,
             sidx0, didx0, sidx1, didx1, rows, acc, gsem):
        cid = lax.axis_index("c")
        sid = lax.axis_index("s")
        wid = cid * _NS + sid
        pltpu.sync_copy(zeros_ref, acc.at[pl.ds(sid * _WB, _WB)])

        @pl.when(sid == _NS - 1)
        def _():
            pltpu.sync_copy(zeros_ref.at[pl.ds(0, _WBL - _WB)],
                            acc.at[pl.ds(_NS * _WB, _WBL - _WB)])

        plsc.subcore_barrier()

        pltpu.sync_copy(src_ref.at[wid, 0], sidx0)
        pltpu.sync_copy(dst_ref.at[wid, 0], didx0)
        for k in range(2):
            pltpu.async_copy(tab_ref.at[sidx0.at[k]], rows.at[k], gsem)

        def outer(b, carry):
            @pl.when(jnp.logical_and(b + 1 < _NBLK, lax.rem(b, 2) == 1))
            def _():
                pltpu.sync_copy(src_ref.at[wid, b + 1], sidx0)
                pltpu.sync_copy(dst_ref.at[wid, b + 1], didx0)

            @pl.when(jnp.logical_and(b + 1 < _NBLK, lax.rem(b, 2) == 0))
            def _():
                pltpu.sync_copy(src_ref.at[wid, b + 1], sidx1)
                pltpu.sync_copy(dst_ref.at[wid, b + 1], didx1)

            def inner(c, carry2):
                g = b * _NBI + c
                gn = g + 2
                bn = lax.div(gn, _NBI)
                cn = lax.rem(gn, _NBI)

                @pl.when(jnp.logical_and(gn < _NSUB, lax.rem(bn, 2) == 0))
                def _():
                    pltpu.async_copy(tab_ref.at[sidx0.at[cn]],
                                     rows.at[lax.rem(gn, 3)], gsem)

                @pl.when(jnp.logical_and(gn < _NSUB, lax.rem(bn, 2) == 1))
                def _():
                    pltpu.async_copy(tab_ref.at[sidx1.at[cn]],
                                     rows.at[lax.rem(gn, 3)], gsem)

                pltpu.make_async_copy(tab_ref.at[pl.ds(0, _SUB)],
                                      rows.at[0], gsem).wait()

                @pl.when(lax.rem(b, 2) == 0)
                def _():
                    pltpu.sync_copy(rows.at[lax.rem(g, 3)],
                                    acc.at[didx0.at[c]], add=True)

                @pl.when(lax.rem(b, 2) == 1)
                def _():
                    pltpu.sync_copy(rows.at[lax.rem(g, 3)],
                                    acc.at[didx1.at[c]], add=True)

                return carry2

            lax.fori_loop(0, _NBI, inner, 0)
            return carry

        lax.fori_loop(0, _NBLK, outer, 0)
        plsc.subcore_barrier()
        pltpu.sync_copy(acc.at[pl.ds(sid * _WB, _WB)],
                        out_ref.at[pl.ds(cid * _N + sid * _WB, _WB)])

        @pl.when(sid == _NS - 1)
        def _():
            pltpu.sync_copy(acc.at[pl.ds(_NS * _WB, _WBL - _WB)],
                            out_ref.at[pl.ds(cid * _N + _NS * _WB,
                                             _WBL - _WB)])

    f = pl.kernel(
        body,
        out_type=jax.ShapeDtypeStruct((_NC * _N, D), jnp.float32),
        mesh=_mesh(),
        scratch_types=[
            pltpu.VMEM((_NBI, _SUB), jnp.int32),
            pltpu.VMEM((_NBI, _SUB), jnp.int32),
            pltpu.VMEM((_NBI, _SUB), jnp.int32),
            pltpu.VMEM((_NBI, _SUB), jnp.int32),
            pltpu.VMEM((3, _SUB, D), jnp.float32),
            pltpu.VMEM_SHARED((_N, D), jnp.float32),
            pltpu.SemaphoreType.DMA,
        ],
    )
    return f(tab, src3, dst3, zeros)


def _decode(z, s3, d3):
    D = 64
    DPAD = z.shape[1]

    def body(z_ref, s_ref, d_ref, out_ref, sidx, didx, zs, zd, outv,
             sem1, sem2):
        cid = lax.axis_index("c")
        sid = lax.axis_index("s")
        wid = cid * _NS + sid
        lane = lax.iota(jnp.int32, 16)
        pltpu.sync_copy(s_ref.at[wid], sidx)
        pltpu.sync_copy(d_ref.at[wid], didx)

        for c in range(_NB - 1):
            pltpu.async_copy(z_ref.at[sidx.at[c]], zs.at[c], sem1)
            pltpu.async_copy(z_ref.at[didx.at[c]], zd.at[c], sem2)

        def sub(c, carry):
            @pl.when(c + _NB - 1 < _NSUB)
            def _():
                nb = (c + _NB - 1) & (_NB - 1)
                pltpu.async_copy(z_ref.at[sidx.at[c + _NB - 1]],
                                 zs.at[nb], sem1)
                pltpu.async_copy(z_ref.at[didx.at[c + _NB - 1]],
                                 zd.at[nb], sem2)

            b = jnp.int32(0) + (c & (_NB - 1))
            pltpu.make_async_copy(z_ref.at[pl.ds(0, _SUB)], zs.at[0],
                                  sem1).wait()
            pltpu.make_async_copy(z_ref.at[pl.ds(0, _SUB)], zd.at[0],
                                  sem2).wait()

            for g in range(_SUB // 16):
                r16 = lane + g * 16
                bvec = jnp.full((16,), 0, jnp.int32) + b
                acc = jnp.zeros((16,), jnp.float32)
                for jj in range(D):
                    col = jnp.full((16,), jj, jnp.int32)
                    a = plsc.load_gather(zs, [bvec, r16, col])
                    bb = plsc.load_gather(zd, [bvec, r16, col])
                    acc = acc + a * bb
                outv[pl.ds(c * _SUB + g * 16, 16)] = acc
            return carry

        lax.fori_loop(0, _NSUB, sub, 0)
        pltpu.sync_copy(outv, out_ref.at[pl.ds(wid * _EPW, _EPW)])

    f = pl.kernel(
        body,
        out_type=jax.ShapeDtypeStruct((_EL,), jnp.float32),
        mesh=_mesh(),
        scratch_types=[
            pltpu.VMEM((_NSUB, _SUB), jnp.int32),
            pltpu.VMEM((_NSUB, _SUB), jnp.int32),
            pltpu.VMEM((_NB, _SUB, DPAD), jnp.float32),
            pltpu.VMEM((_NB, _SUB, DPAD), jnp.float32),
            pltpu.VMEM((_EPW,), jnp.float32),
            pltpu.SemaphoreType.DMA,
            pltpu.SemaphoreType.DMA,
        ],
        compiler_params=pltpu.CompilerParams(needs_layout_passes=False),
    )
    return f(z, s3, d3)


def _mm_body(x_ref, w_ref, o_ref):
    o_ref[...] = jnp.dot(x_ref[...], w_ref[...],
                         preferred_element_type=jnp.float32)


def _tc_mm(x, w):
    return pl.pallas_call(
        _mm_body,
        out_shape=jax.ShapeDtypeStruct((x.shape[0], w.shape[1]), jnp.float32),
    )(x, w)


def _mid_body(y0_ref, m_ref, eps_ref, b11_ref, w12_ref, b12_ref, w21_ref,
              o_ref):
    m = m_ref[0:_N, :] + m_ref[_N:2 * _N, :]
    pre = (1.0 + eps_ref[0, 0]) * y0_ref[...] + m + b11_ref[...]
    a = jnp.maximum(pre, 0.0)
    h1 = jnp.maximum(
        jnp.dot(a, w12_ref[...], preferred_element_type=jnp.float32)
        + b12_ref[...], 0.0)
    o_ref[...] = jnp.dot(h1, w21_ref[...], preferred_element_type=jnp.float32)


def _tc_mid(y0, m0, eps1, b11, W12, b12, W21):
    return pl.pallas_call(
        _mid_body,
        out_shape=jax.ShapeDtypeStruct((_N, W21.shape[1]), jnp.float32),
    )(y0, m0, eps1, b11, W12, b12, W21)


def _out_body(y1_ref, m_ref, eps_ref, b21_ref, w22_ref, b22_ref, o_ref):
    m = m_ref[0:_N, :] + m_ref[_N:2 * _N, :]
    pre = (1.0 + eps_ref[0, 0]) * y1_ref[...] + m + b21_ref[...]
    a = jnp.maximum(pre, 0.0)
    o_ref[...] = jnp.dot(a, w22_ref[...],
                         preferred_element_type=jnp.float32) + b22_ref[...]


def _tc_out(y1, m1, eps2, b21, W22, b22):
    return pl.pallas_call(
        _out_body,
        out_shape=jax.ShapeDtypeStruct((_N, W22.shape[1]), jnp.float32),
    )(y1, m1, eps2, b21, W22, b22)


def kernel(x, edge_index, edge_label_index, eps1, W11, b11, W12, b12,
           eps2, W21, b21, W22, b22):
    src4 = edge_index[0].astype(jnp.int32).reshape(_NW, _NBLK, _NBI, _SUB)
    dst4 = edge_index[1].astype(jnp.int32).reshape(_NW, _NBLK, _NBI, _SUB)
    s3 = edge_label_index[0].astype(jnp.int32).reshape(_NW, _NSUB, _SUB)
    d3 = edge_label_index[1].astype(jnp.int32).reshape(_NW, _NSUB, _SUB)

    e1 = jnp.reshape(eps1, (1, 1)).astype(jnp.float32)
    e2 = jnp.reshape(eps2, (1, 1)).astype(jnp.float32)
    b11r = jnp.reshape(b11, (1, -1))
    b12r = jnp.reshape(b12, (1, -1))
    b21r = jnp.reshape(b21, (1, -1))
    b22r = jnp.reshape(b22, (1, -1))

    zeros128 = jnp.zeros((_WB, 128), jnp.float32)
    W21p = jnp.concatenate([W21, jnp.zeros((128, 64), jnp.float32)], axis=1)
    b21p = jnp.concatenate([b21r, jnp.zeros((1, 64), jnp.float32)], axis=1)
    W22p = jnp.zeros((128, 128), jnp.float32).at[:64, :64].set(W22)
    b22p = jnp.concatenate([b22r, jnp.zeros((1, 64), jnp.float32)], axis=1)

    y0 = _tc_mm(x, W11)
    m0 = _segsum(y0, src4, dst4, zeros128)
    y1 = _tc_mid(y0, m0, e1, b11r, W12, b12r, W21p)
    m1 = _segsum(y1, src4, dst4, zeros128)
    z = _tc_out(y1, m1, e2, b21p, W22p, b22p)
    logits = _decode(z, s3, d3)
    return logits


# speedup vs baseline: 2.3165x; 2.3165x over previous
import functools

import jax
import jax.numpy as jnp
from jax import lax
from jax.experimental import pallas as pl
from jax.experimental.pallas import tpu as pltpu
from jax.experimental.pallas import tpu_sc as plsc

_N = 10000
_E = 320000
_EL = 320000

_NC = 2
_NS = 16
_NW = _NC * _NS

_SUB = 80
_EPW = _E // _NW
_NSUB = _EPW // _SUB
_WB = 624
_WBL = _N - 15 * _WB
_NB = 4
_NBI = 25
_NBLK = _NSUB // _NBI


def _mesh():
    return plsc.VectorSubcoreMesh(
        core_axis_name="c", subcore_axis_name="s",
        num_cores=_NC, num_subcores=_NS)


def _segsum(tab, src3, dst3, zeros):
    D = tab.shape[1]

    def body(tab_ref, src_ref, dst_ref, zeros_ref, out_ref,
             sidx0, didx0, sidx1, didx1, rows, acc, gsem):
        cid = lax.axis_index("c")
        sid = lax.axis_index("s")
        wid = cid * _NS + sid
        pltpu.sync_copy(zeros_ref, acc.at[pl.ds(sid * _WB, _WB)])

        @pl.when(sid == _NS - 1)
        def _():
            pltpu.sync_copy(zeros_ref.at[pl.ds(0, _WBL - _WB)],
                            acc.at[pl.ds(_NS * _WB, _WBL - _WB)])

        plsc.subcore_barrier()

        pltpu.sync_copy(src_ref.at[wid, 0], sidx0)
        pltpu.sync_copy(dst_ref.at[wid, 0], didx0)
        for k in range(2):
            pltpu.async_copy(tab_ref.at[sidx0.at[k]], rows.at[k], gsem)

        def outer(b, carry):
            @pl.when(jnp.logical_and(b + 1 < _NBLK, lax.rem(b, 2) == 1))
            def _():
                pltpu.sync_copy(src_ref.at[wid, b + 1], sidx0)
                pltpu.sync_copy(dst_ref.at[wid, b + 1], didx0)

            @pl.when(jnp.logical_and(b + 1 < _NBLK, lax.rem(b, 2) == 0))
            def _():
                pltpu.sync_copy(src_ref.at[wid, b + 1], sidx1)
                pltpu.sync_copy(dst_ref.at[wid, b + 1], didx1)

            def inner(c, carry2):
                g = b * _NBI + c
                gn = g + 2
                bn = lax.div(gn, _NBI)
                cn = lax.rem(gn, _NBI)

                @pl.when(jnp.logical_and(gn < _NSUB, lax.rem(bn, 2) == 0))
                def _():
                    pltpu.async_copy(tab_ref.at[sidx0.at[cn]],
                                     rows.at[lax.rem(gn, 3)], gsem)

                @pl.when(jnp.logical_and(gn < _NSUB, lax.rem(bn, 2) == 1))
                def _():
                    pltpu.async_copy(tab_ref.at[sidx1.at[cn]],
                                     rows.at[lax.rem(gn, 3)], gsem)

                pltpu.make_async_copy(tab_ref.at[pl.ds(0, _SUB)],
                                      rows.at[0], gsem).wait()

                @pl.when(lax.rem(b, 2) == 0)
                def _():
                    pltpu.sync_copy(rows.at[lax.rem(g, 3)],
                                    acc.at[didx0.at[c]], add=True)

                @pl.when(lax.rem(b, 2) == 1)
                def _():
                    pltpu.sync_copy(rows.at[lax.rem(g, 3)],
                                    acc.at[didx1.at[c]], add=True)

                return carry2

            lax.fori_loop(0, _NBI, inner, 0)
            return carry

        lax.fori_loop(0, _NBLK, outer, 0)
        plsc.subcore_barrier()
        pltpu.sync_copy(acc.at[pl.ds(sid * _WB, _WB)],
                        out_ref.at[pl.ds(cid * _N + sid * _WB, _WB)])

        @pl.when(sid == _NS - 1)
        def _():
            pltpu.sync_copy(acc.at[pl.ds(_NS * _WB, _WBL - _WB)],
                            out_ref.at[pl.ds(cid * _N + _NS * _WB,
                                             _WBL - _WB)])

    f = pl.kernel(
        body,
        out_type=jax.ShapeDtypeStruct((_NC * _N, D), jnp.float32),
        mesh=_mesh(),
        scratch_types=[
            pltpu.VMEM((_NBI, _SUB), jnp.int32),
            pltpu.VMEM((_NBI, _SUB), jnp.int32),
            pltpu.VMEM((_NBI, _SUB), jnp.int32),
            pltpu.VMEM((_NBI, _SUB), jnp.int32),
            pltpu.VMEM((3, _SUB, D), jnp.float32),
            pltpu.VMEM_SHARED((_N, D), jnp.float32),
            pltpu.SemaphoreType.DMA,
        ],
    )
    return f(tab, src3, dst3, zeros)


def _decode(z, s3, d3):
    D = 64
    DPAD = z.shape[1]

    def body(z_ref, s_ref, d_ref, out_ref, sidx, didx, zs, zd, outv,
             buf, sem1, sem2):
        cid = lax.axis_index("c")
        sid = lax.axis_index("s")
        wid = cid * _NS + sid
        lane = lax.iota(jnp.int32, 16)
        for off in (16, 40, 64, 80):
            buf[pl.ds(off, 16)] = jnp.zeros((16,), jnp.float32)
        pltpu.sync_copy(s_ref.at[wid], sidx)
        pltpu.sync_copy(d_ref.at[wid], didx)

        for c in range(_NB - 1):
            pltpu.async_copy(z_ref.at[sidx.at[c]], zs.at[c], sem1)
            pltpu.async_copy(z_ref.at[didx.at[c]], zd.at[c], sem2)

        def sub(c, carry):
            @pl.when(c + _NB - 1 < _NSUB)
            def _():
                nb = (c + _NB - 1) & (_NB - 1)
                pltpu.async_copy(z_ref.at[sidx.at[c + _NB - 1]],
                                 zs.at[nb], sem1)
                pltpu.async_copy(z_ref.at[didx.at[c + _NB - 1]],
                                 zd.at[nb], sem2)

            b = jnp.int32(0) + (c & (_NB - 1))
            pltpu.make_async_copy(z_ref.at[pl.ds(0, _SUB)], zs.at[0],
                                  sem1).wait()
            pltpu.make_async_copy(z_ref.at[pl.ds(0, _SUB)], zd.at[0],
                                  sem2).wait()

            for g in range(_SUB // 16):
                def quad(it, tot):
                    for k in range(4):
                        t = it * 4 + k
                        e = g * 16 + t
                        p = zs[b, e, pl.ds(0, 16)] * zd[b, e, pl.ds(0, 16)]
                        for q in range(1, D // 16):
                            p = p + (zs[b, e, pl.ds(q * 16, 16)]
                                     * zd[b, e, pl.ds(q * 16, 16)])
                        buf[pl.ds(24 * k, 16)] = p
                        w = p + buf[pl.ds(24 * k + 8, 16)]
                        s = (((w[0] + w[1]) + (w[2] + w[3]))
                             + ((w[4] + w[5]) + (w[6] + w[7])))
                        tot = jnp.where(lane == t, s, tot)
                    return tot

                tot = lax.fori_loop(0, 4, quad,
                                    jnp.zeros((16,), jnp.float32))
                outv[pl.ds(c * _SUB + g * 16, 16)] = tot
            return carry

        lax.fori_loop(0, _NSUB, sub, 0)
        pltpu.sync_copy(outv, out_ref.at[pl.ds(wid * _EPW, _EPW)])

    f = pl.kernel(
        body,
        out_type=jax.ShapeDtypeStruct((_EL,), jnp.float32),
        mesh=_mesh(),
        scratch_types=[
            pltpu.VMEM((_NSUB, _SUB), jnp.int32),
            pltpu.VMEM((_NSUB, _SUB), jnp.int32),
            pltpu.VMEM((_NB, _SUB, DPAD), jnp.float32),
            pltpu.VMEM((_NB, _SUB, DPAD), jnp.float32),
            pltpu.VMEM((_EPW,), jnp.float32),
            pltpu.VMEM((96,), jnp.float32),
            pltpu.SemaphoreType.DMA,
            pltpu.SemaphoreType.DMA,
        ],
        compiler_params=pltpu.CompilerParams(needs_layout_passes=False),
    )
    return f(z, s3, d3)


def _mm_body(x_ref, w_ref, o_ref):
    o_ref[...] = jnp.dot(x_ref[...], w_ref[...],
                         preferred_element_type=jnp.float32)


def _tc_mm(x, w):
    return pl.pallas_call(
        _mm_body,
        out_shape=jax.ShapeDtypeStruct((x.shape[0], w.shape[1]), jnp.float32),
    )(x, w)


def _mid_body(y0_ref, m_ref, eps_ref, b11_ref, w12_ref, b12_ref, w21_ref,
              o_ref):
    m = m_ref[0:_N, :] + m_ref[_N:2 * _N, :]
    pre = (1.0 + eps_ref[0, 0]) * y0_ref[...] + m + b11_ref[...]
    a = jnp.maximum(pre, 0.0)
    h1 = jnp.maximum(
        jnp.dot(a, w12_ref[...], preferred_element_type=jnp.float32)
        + b12_ref[...], 0.0)
    o_ref[...] = jnp.dot(h1, w21_ref[...], preferred_element_type=jnp.float32)


def _tc_mid(y0, m0, eps1, b11, W12, b12, W21):
    return pl.pallas_call(
        _mid_body,
        out_shape=jax.ShapeDtypeStruct((_N, W21.shape[1]), jnp.float32),
    )(y0, m0, eps1, b11, W12, b12, W21)


def _out_body(y1_ref, m_ref, eps_ref, b21_ref, w22_ref, b22_ref, o_ref):
    m = m_ref[0:_N, :] + m_ref[_N:2 * _N, :]
    pre = (1.0 + eps_ref[0, 0]) * y1_ref[...] + m + b21_ref[...]
    a = jnp.maximum(pre, 0.0)
    o_ref[...] = jnp.dot(a, w22_ref[...],
                         preferred_element_type=jnp.float32) + b22_ref[...]


def _tc_out(y1, m1, eps2, b21, W22, b22):
    return pl.pallas_call(
        _out_body,
        out_shape=jax.ShapeDtypeStruct((_N, W22.shape[1]), jnp.float32),
    )(y1, m1, eps2, b21, W22, b22)


def kernel(x, edge_index, edge_label_index, eps1, W11, b11, W12, b12,
           eps2, W21, b21, W22, b22):
    src4 = edge_index[0].astype(jnp.int32).reshape(_NW, _NBLK, _NBI, _SUB)
    dst4 = edge_index[1].astype(jnp.int32).reshape(_NW, _NBLK, _NBI, _SUB)
    s3 = edge_label_index[0].astype(jnp.int32).reshape(_NW, _NSUB, _SUB)
    d3 = edge_label_index[1].astype(jnp.int32).reshape(_NW, _NSUB, _SUB)

    e1 = jnp.reshape(eps1, (1, 1)).astype(jnp.float32)
    e2 = jnp.reshape(eps2, (1, 1)).astype(jnp.float32)
    b11r = jnp.reshape(b11, (1, -1))
    b12r = jnp.reshape(b12, (1, -1))
    b21r = jnp.reshape(b21, (1, -1))
    b22r = jnp.reshape(b22, (1, -1))

    zeros128 = jnp.zeros((_WB, 128), jnp.float32)
    W21p = jnp.concatenate([W21, jnp.zeros((128, 64), jnp.float32)], axis=1)
    b21p = jnp.concatenate([b21r, jnp.zeros((1, 64), jnp.float32)], axis=1)
    W22p = jnp.zeros((128, 128), jnp.float32).at[:64, :64].set(W22)
    b22p = jnp.concatenate([b22r, jnp.zeros((1, 64), jnp.float32)], axis=1)

    y0 = _tc_mm(x, W11)
    m0 = _segsum(y0, src4, dst4, zeros128)
    y1 = _tc_mid(y0, m0, e1, b11r, W12, b12r, W21p)
    m1 = _segsum(y1, src4, dst4, zeros128)
    z = _tc_out(y1, m1, e2, b21p, W22p, b22p)
    logits = _decode(z, s3, d3)
    return logits
